# baseline (speedup 1.0000x reference)
import jax
import jax.numpy as jnp
from jax.experimental import pallas as pl

N = 10000
D = 128
H = 32
BLOCK = 1000


def _cell_kernel(x_ref, h_ref, c_ref, wx_ref, wh_ref, b_ref,
                 wci_ref, wcf_ref, wco_ref, wlin_ref, blin_ref,
                 out_ref, h0_ref, cn_ref):
    g = jnp.dot(x_ref[...], wx_ref[...], preferred_element_type=jnp.float32)
    g = g + jnp.dot(h_ref[...], wh_ref[...], preferred_element_type=jnp.float32)
    g = g + b_ref[...]
    c = c_ref[...]
    i = jax.nn.sigmoid(g[:, 0 * H:1 * H] + wci_ref[...] * c)
    f = jax.nn.sigmoid(g[:, 1 * H:2 * H] + wcf_ref[...] * c)
    t = jnp.tanh(g[:, 2 * H:3 * H])
    cn = f * c + i * t
    o = jax.nn.sigmoid(g[:, 3 * H:4 * H] + wco_ref[...] * cn)
    h0 = o * jnp.tanh(cn)
    hr = jnp.maximum(h0, 0.0)
    out_ref[...] = jnp.sum(hr * wlin_ref[...], axis=1, keepdims=True) + blin_ref[...]
    h0_ref[...] = h0
    cn_ref[...] = cn


def kernel(x, edge_index, edge_weight, h, c,
           W_xi, b_xi, W_hi, b_hi, w_ci, b_i,
           W_xf, b_xf, W_hf, b_hf, w_cf, b_f,
           W_xc, b_xc, W_hc, b_hc, b_c,
           W_xo, b_xo, W_ho, b_ho, w_co, b_o,
           W_lin, b_lin):
    wx = jnp.concatenate([W_xi, W_xf, W_xc, W_xo], axis=1)
    wh = jnp.concatenate([W_hi, W_hf, W_hc, W_ho], axis=1)
    bias = jnp.concatenate([b_xi + b_hi + b_i, b_xf + b_hf + b_f,
                            b_xc + b_hc + b_c, b_xo + b_ho + b_o])
    bias = bias.reshape(1, 4 * H)
    wci = w_ci.reshape(1, H)
    wcf = w_cf.reshape(1, H)
    wco = w_co.reshape(1, H)
    wlin = W_lin.reshape(1, H)
    blin = b_lin.reshape(1, 1)

    grid = (N // BLOCK,)
    row = lambda i: (i, 0)
    fixed = lambda i: (0, 0)
    out, h0, cn = pl.pallas_call(
        _cell_kernel,
        grid=grid,
        in_specs=[
            pl.BlockSpec((BLOCK, D), row),
            pl.BlockSpec((BLOCK, H), row),
            pl.BlockSpec((BLOCK, H), row),
            pl.BlockSpec((D, 4 * H), fixed),
            pl.BlockSpec((H, 4 * H), fixed),
            pl.BlockSpec((1, 4 * H), fixed),
            pl.BlockSpec((1, H), fixed),
            pl.BlockSpec((1, H), fixed),
            pl.BlockSpec((1, H), fixed),
            pl.BlockSpec((1, H), fixed),
            pl.BlockSpec((1, 1), fixed),
        ],
        out_specs=[
            pl.BlockSpec((BLOCK, 1), row),
            pl.BlockSpec((BLOCK, H), row),
            pl.BlockSpec((BLOCK, H), row),
        ],
        out_shape=[
            jax.ShapeDtypeStruct((N, 1), jnp.float32),
            jax.ShapeDtypeStruct((N, H), jnp.float32),
            jax.ShapeDtypeStruct((N, H), jnp.float32),
        ],
    )(x, h, c, wx, wh, bias, wci, wcf, wco, wlin, blin)
    return (out, h0, cn)


# speedup vs baseline: 1.0470x; 1.0470x over previous
import jax
import jax.numpy as jnp
from jax.experimental import pallas as pl

N = 10000
D = 128
H = 32
BLOCK = 2000


def _cell_kernel(x_ref, h_ref, c_ref, wx_ref, wh_ref, b_ref,
                 wc2_ref, wco_ref, wlin_ref, blin_ref,
                 out_ref, h0_ref, cn_ref):
    g = jnp.dot(x_ref[...], wx_ref[...], preferred_element_type=jnp.float32)
    g = g + jnp.dot(h_ref[...], wh_ref[...], preferred_element_type=jnp.float32)
    g = g + b_ref[...]
    c = c_ref[...]
    z = jnp.zeros_like(c)
    c4 = jnp.concatenate([c, c, z, z], axis=1)
    pre = g + wc2_ref[...] * c4
    grp = jax.lax.broadcasted_iota(jnp.int32, (1, 4 * H), 1) // H
    is_t = (grp == 2)
    alpha = jnp.where(is_t, 2.0, 1.0).astype(jnp.float32)
    beta = jnp.where(is_t, -1.0, 0.0).astype(jnp.float32)
    s = jax.nn.sigmoid(pre * alpha)
    act = s * alpha + beta
    i = act[:, 0 * H:1 * H]
    f = act[:, 1 * H:2 * H]
    t = act[:, 2 * H:3 * H]
    cn = f * c + i * t
    opre = pre[:, 3 * H:4 * H] + wco_ref[...] * cn
    packed = jnp.concatenate([opre, 2.0 * cn], axis=1)
    sp = jax.nn.sigmoid(packed)
    o = sp[:, 0 * H:1 * H]
    tcn = 2.0 * sp[:, 1 * H:2 * H] - 1.0
    h0 = o * tcn
    hr = jnp.maximum(h0, 0.0)
    out_ref[...] = jnp.sum(hr * wlin_ref[...], axis=1, keepdims=True) + blin_ref[...]
    h0_ref[...] = h0
    cn_ref[...] = cn


def kernel(x, edge_index, edge_weight, h, c,
           W_xi, b_xi, W_hi, b_hi, w_ci, b_i,
           W_xf, b_xf, W_hf, b_hf, w_cf, b_f,
           W_xc, b_xc, W_hc, b_hc, b_c,
           W_xo, b_xo, W_ho, b_ho, w_co, b_o,
           W_lin, b_lin):
    wx = jnp.concatenate([W_xi, W_xf, W_xc, W_xo], axis=1)
    wh = jnp.concatenate([W_hi, W_hf, W_hc, W_ho], axis=1)
    bias = jnp.concatenate([b_xi + b_hi + b_i, b_xf + b_hf + b_f,
                            b_xc + b_hc + b_c, b_xo + b_ho + b_o])
    bias = bias.reshape(1, 4 * H)
    zH = jnp.zeros((H,), jnp.float32)
    wc2 = jnp.concatenate([w_ci, w_cf, zH, zH]).reshape(1, 4 * H)
    wco = w_co.reshape(1, H)
    wlin = W_lin.reshape(1, H)
    blin = b_lin.reshape(1, 1)

    grid = (N // BLOCK,)
    row = lambda i: (i, 0)
    fixed = lambda i: (0, 0)
    out, h0, cn = pl.pallas_call(
        _cell_kernel,
        grid=grid,
        in_specs=[
            pl.BlockSpec((BLOCK, D), row),
            pl.BlockSpec((BLOCK, H), row),
            pl.BlockSpec((BLOCK, H), row),
            pl.BlockSpec((D, 4 * H), fixed),
            pl.BlockSpec((H, 4 * H), fixed),
            pl.BlockSpec((1, 4 * H), fixed),
            pl.BlockSpec((1, 4 * H), fixed),
            pl.BlockSpec((1, H), fixed),
            pl.BlockSpec((1, H), fixed),
            pl.BlockSpec((1, 1), fixed),
        ],
        out_specs=[
            pl.BlockSpec((BLOCK, 1), row),
            pl.BlockSpec((BLOCK, H), row),
            pl.BlockSpec((BLOCK, H), row),
        ],
        out_shape=[
            jax.ShapeDtypeStruct((N, 1), jnp.float32),
            jax.ShapeDtypeStruct((N, H), jnp.float32),
            jax.ShapeDtypeStruct((N, H), jnp.float32),
        ],
    )(x, h, c, wx, wh, bias, wc2, wco, wlin, blin)
    return (out, h0, cn)
